# Initial kernel scaffold; baseline (speedup 1.0000x reference)
import jax
import jax.numpy as jnp
from jax.experimental import pallas as pl


def kernel(tag_ids, table, W1, b1, W2, b2):
    raise NotImplementedError("write your pallas kernel here")


# speedup vs baseline: 12.3456x; 12.3456x over previous
import functools

import jax
import jax.numpy as jnp
from jax import lax
from jax.experimental import pallas as pl
from jax.experimental.pallas import tpu as pltpu
from jax.experimental.pallas import tpu_sc as plsc

B = 16384
T = 50
D = 64
OUT = 128
LANES = 16
NC = 2
NS = 16
NW = NC * NS
ROWS_PER_W = B // NW
RB = 8
STEPS = ROWS_PER_W // RB
IDX_CH = 80
NG = (RB * T) // IDX_CH


def _make_sc_pool():
    mesh = plsc.VectorSubcoreMesh(core_axis_name="c", subcore_axis_name="s")

    @functools.partial(
        pl.kernel,
        out_type=jax.ShapeDtypeStruct((B, D), jnp.float32),
        mesh=mesh,
        compiler_params=pltpu.CompilerParams(use_tc_tiling_on_sc=False),
        scratch_types=[
            pltpu.VMEM((RB * T,), jnp.int32),
            pltpu.VMEM((RB * T, D), jnp.float32),
            pltpu.VMEM((RB, D), jnp.float32),
            pltpu.SemaphoreType.DMA,
        ],
    )
    def sc_pool(ids_hbm, table_hbm, out_hbm, idx_v, rows_v, acc_v, sem):
        wid = lax.axis_index("s") * NC + lax.axis_index("c")
        idx_base = wid * ROWS_PER_W * T
        out_base = wid * ROWS_PER_W

        @pl.loop(0, STEPS)
        def _(step):
            pltpu.sync_copy(
                ids_hbm.at[pl.ds(idx_base + step * (RB * T), RB * T)], idx_v)
            copies = []
            for g in range(NG):
                copies.append(pltpu.async_copy(
                    table_hbm.at[idx_v.at[pl.ds(g * IDX_CH, IDX_CH)]],
                    rows_v.at[pl.ds(g * IDX_CH, IDX_CH)],
                    sem,
                ))
            for c in copies:
                c.wait()

            @pl.loop(0, RB)
            def _(r):
                row0 = r * T

                def body(t, acc):
                    return tuple(
                        acc[j] + rows_v[row0 + t, pl.ds(j * LANES, LANES)]
                        for j in range(D // LANES)
                    )

                init = tuple(
                    rows_v[row0, pl.ds(j * LANES, LANES)]
                    for j in range(D // LANES)
                )
                acc = lax.fori_loop(1, T, body, init)
                for j in range(D // LANES):
                    acc_v[r, pl.ds(j * LANES, LANES)] = acc[j]

            pltpu.sync_copy(
                acc_v, out_hbm.at[pl.ds(out_base + step * RB, RB)])

    return sc_pool


_sc_pool_cache = []


def _sc_pool(ids2d, table):
    if not _sc_pool_cache:
        _sc_pool_cache.append(_make_sc_pool())
    return _sc_pool_cache[0](ids2d, table)

RBLK = 1024


def _mlp_body(ids_ref, ps_ref, w1_ref, b1_ref, w2_ref, b2_ref, out_ref):
    ids = ids_ref[...]
    cnt = jnp.sum((ids != 0).astype(jnp.float32), axis=1, keepdims=True)
    denom = jnp.maximum(cnt, 1.0)
    pooled = ps_ref[...] / denom
    h = jnp.maximum(
        jnp.dot(pooled, w1_ref[...], preferred_element_type=jnp.float32)
        + b1_ref[...], 0.0)
    out_ref[...] = (
        jnp.dot(h, w2_ref[...], preferred_element_type=jnp.float32)
        + b2_ref[...])


def _tc_mlp(tag_ids, pooled_sum, W1, b1, W2, b2):
    return pl.pallas_call(
        _mlp_body,
        grid=(B // RBLK,),
        in_specs=[
            pl.BlockSpec((RBLK, T), lambda i: (i, 0)),
            pl.BlockSpec((RBLK, D), lambda i: (i, 0)),
            pl.BlockSpec((D, D), lambda i: (0, 0)),
            pl.BlockSpec((1, D), lambda i: (0, 0)),
            pl.BlockSpec((D, OUT), lambda i: (0, 0)),
            pl.BlockSpec((1, OUT), lambda i: (0, 0)),
        ],
        out_specs=pl.BlockSpec((RBLK, OUT), lambda i: (i, 0)),
        out_shape=jax.ShapeDtypeStruct((B, OUT), jnp.float32),
    )(tag_ids, pooled_sum, W1, b1, W2, b2)


def kernel(tag_ids, table, W1, b1, W2, b2):
    ids_flat = tag_ids.reshape(B * T)
    pooled_sum = _sc_pool(ids_flat, table)
    return _tc_mlp(tag_ids, pooled_sum, W1, b1.reshape(1, D), W2,
                   b2.reshape(1, OUT))


# speedup vs baseline: 21.0423x; 1.7044x over previous
import functools

import jax
import jax.numpy as jnp
from jax import lax
from jax.experimental import pallas as pl
from jax.experimental.pallas import tpu as pltpu
from jax.experimental.pallas import tpu_sc as plsc

B = 16384
T = 50
D = 64
OUT = 128
LANES = 16
NC = 2
NS = 16
NW = NC * NS
ROWS_PER_W = B // NW
RB = 8
STEPS = ROWS_PER_W // RB
IDX_CH = 80
NG = (RB * T) // IDX_CH
NJ = D // LANES


def _make_sc_pool():
    mesh = plsc.VectorSubcoreMesh(core_axis_name="c", subcore_axis_name="s")

    @functools.partial(
        pl.kernel,
        out_type=jax.ShapeDtypeStruct((B, D), jnp.float32),
        mesh=mesh,
        compiler_params=pltpu.CompilerParams(use_tc_tiling_on_sc=False),
        scratch_types=[
            pltpu.VMEM((ROWS_PER_W * T,), jnp.int32),
            pltpu.VMEM((RB * T, D), jnp.float32),
            pltpu.VMEM((RB * T, D), jnp.float32),
            pltpu.VMEM((RB, D), jnp.float32),
            pltpu.VMEM((RB, D), jnp.float32),
            pltpu.SemaphoreType.DMA,
            pltpu.SemaphoreType.DMA,
            pltpu.SemaphoreType.DMA,
            pltpu.SemaphoreType.DMA,
        ],
    )
    def sc_pool(ids_hbm, table_hbm, out_hbm, idx_all, rows0, rows1,
                outv0, outv1, semg0, semg1, semo0, semo1):
        wid = lax.axis_index("s") * NC + lax.axis_index("c")
        idx_base = wid * ROWS_PER_W * T
        out_base = wid * ROWS_PER_W

        pltpu.sync_copy(
            ids_hbm.at[pl.ds(idx_base, ROWS_PER_W * T)], idx_all)

        def fire(rows_ref, sem, s):
            for g in range(NG):
                pltpu.make_async_copy(
                    table_hbm.at[idx_all.at[
                        pl.ds(s * (RB * T) + g * IDX_CH, IDX_CH)]],
                    rows_ref.at[pl.ds(g * IDX_CH, IDX_CH)],
                    sem,
                ).start()

        def drain(rows_ref, sem):
            for g in range(NG):
                pltpu.make_async_copy(
                    table_hbm.at[idx_all.at[pl.ds(g * IDX_CH, IDX_CH)]],
                    rows_ref.at[pl.ds(g * IDX_CH, IDX_CH)],
                    sem,
                ).wait()

        def out_copy(out_v, semo, s):
            return pltpu.make_async_copy(
                out_v, out_hbm.at[pl.ds(out_base + s * RB, RB)], semo)

        def process(rows_ref, out_v, semo, s, wait_from):
            drain(rows_ref, [semg0, semg1][wait_from])

            @pl.when(s >= 2)
            def _():
                out_copy(out_v, semo, s).wait()

            @pl.loop(0, RB)
            def _(r):
                row0 = r * T

                def body(t, acc):
                    t2 = row0 + 2 * t
                    return tuple(
                        acc[j]
                        + rows_ref[t2, pl.ds(j * LANES, LANES)]
                        + rows_ref[t2 + 1, pl.ds(j * LANES, LANES)]
                        for j in range(NJ)
                    )

                init = tuple(
                    rows_ref[row0 + T - 2, pl.ds(j * LANES, LANES)]
                    + rows_ref[row0 + T - 1, pl.ds(j * LANES, LANES)]
                    for j in range(NJ)
                )
                acc = lax.fori_loop(0, (T - 2) // 2, body, init)
                for j in range(NJ):
                    out_v[r, pl.ds(j * LANES, LANES)] = acc[j]

            out_copy(out_v, semo, s).start()

        fire(rows0, semg0, 0)
        fire(rows1, semg1, 1)

        @pl.loop(0, STEPS, step=2)
        def _(s):
            process(rows0, outv0, semo0, s, 0)

            @pl.when(s + 2 < STEPS)
            def _():
                fire(rows0, semg0, s + 2)
            process(rows1, outv1, semo1, s + 1, 1)

            @pl.when(s + 3 < STEPS)
            def _():
                fire(rows1, semg1, s + 3)

        out_copy(outv0, semo0, 0).wait()
        out_copy(outv1, semo1, 0).wait()

    return sc_pool


_sc_pool_cache = []


def _sc_pool(ids_flat, table):
    if not _sc_pool_cache:
        _sc_pool_cache.append(_make_sc_pool())
    return _sc_pool_cache[0](ids_flat, table)


RBLK = 1024


def _mlp_body(ids_ref, ps_ref, w1_ref, b1_ref, w2_ref, b2_ref, out_ref):
    ids = ids_ref[...]
    cnt = jnp.sum((ids != 0).astype(jnp.float32), axis=1, keepdims=True)
    denom = jnp.maximum(cnt, 1.0)
    pooled = ps_ref[...] / denom
    h = jnp.maximum(
        jnp.dot(pooled, w1_ref[...], preferred_element_type=jnp.float32)
        + b1_ref[...], 0.0)
    out_ref[...] = (
        jnp.dot(h, w2_ref[...], preferred_element_type=jnp.float32)
        + b2_ref[...])


def _tc_mlp(tag_ids, pooled_sum, W1, b1, W2, b2):
    return pl.pallas_call(
        _mlp_body,
        grid=(B // RBLK,),
        in_specs=[
            pl.BlockSpec((RBLK, T), lambda i: (i, 0)),
            pl.BlockSpec((RBLK, D), lambda i: (i, 0)),
            pl.BlockSpec((D, D), lambda i: (0, 0)),
            pl.BlockSpec((1, D), lambda i: (0, 0)),
            pl.BlockSpec((D, OUT), lambda i: (0, 0)),
            pl.BlockSpec((1, OUT), lambda i: (0, 0)),
        ],
        out_specs=pl.BlockSpec((RBLK, OUT), lambda i: (i, 0)),
        out_shape=jax.ShapeDtypeStruct((B, OUT), jnp.float32),
    )(tag_ids, pooled_sum, W1, b1, W2, b2)


def kernel(tag_ids, table, W1, b1, W2, b2):
    ids_flat = tag_ids.reshape(B * T)
    pooled_sum = _sc_pool(ids_flat, table)
    return _tc_mlp(tag_ids, pooled_sum, W1, b1.reshape(1, D), W2,
                   b2.reshape(1, OUT))
